# baseline (speedup 1.0000x reference)
import functools

import jax
import jax.numpy as jnp
from jax import lax
from jax.experimental import pallas as pl
from jax.experimental.pallas import tpu as pltpu
from jax.experimental.pallas import tpu_sc as plsc

_NC, _NS = 2, 16
_NW = _NC * _NS


def _vq_idx_body(z_ref, cbT_ref, cb_ref, idx_ref):
    zb = z_ref[...]
    cbT = cbT_ref[...]
    cb = cb_ref[...]
    K = cb.shape[0]
    BT = zb.shape[0]
    cross = lax.dot_general(
        zb, cbT, (((1,), (0,)), ((), ())),
        preferred_element_type=jnp.float32)
    z_sq = jnp.sum(zb * zb, axis=-1, keepdims=True)
    e_sq = jnp.sum(cb * cb, axis=-1)
    dists = z_sq - 2.0 * cross + e_sq[None, :]
    m = jnp.min(dists, axis=-1, keepdims=True)
    kiota = lax.broadcasted_iota(jnp.int32, (BT, K), 1)
    idx_ref[...] = jnp.min(jnp.where(dists == m, kiota, K), axis=-1)


def _sc_gather(codebook_pad, idx_flat, N, D, DP):
    b_per_w = N // _NW
    mesh = plsc.VectorSubcoreMesh(core_axis_name="c", subcore_axis_name="s")

    @functools.partial(
        pl.kernel,
        mesh=mesh,
        out_type=jax.ShapeDtypeStruct((N, DP), jnp.float32),
        scratch_types=[
            pltpu.VMEM((b_per_w,), jnp.int32),
            pltpu.VMEM((b_per_w, DP), jnp.float32),
            pltpu.SemaphoreType.DMA,
        ],
    )
    def gk(table_hbm, idx_hbm, out_hbm, idx_v, rows_v, sem):
        wid = lax.axis_index("s") * _NC + lax.axis_index("c")
        base = wid * b_per_w
        pltpu.sync_copy(idx_hbm.at[pl.ds(base, b_per_w)], idx_v)
        pltpu.async_copy(table_hbm.at[idx_v], rows_v, sem).wait()
        pltpu.sync_copy(rows_v, out_hbm.at[pl.ds(base, b_per_w)])

    return gk(codebook_pad, idx_flat)


def kernel(z_e, codebook):
    B, T, D = z_e.shape
    K = codebook.shape[0]
    N = B * T
    z = z_e.reshape(N, D)
    cbT = codebook.T
    BT = 512

    idx_flat = pl.pallas_call(
        _vq_idx_body,
        grid=(N // BT,),
        in_specs=[
            pl.BlockSpec((BT, D), lambda i: (i, 0)),
            pl.BlockSpec((D, K), lambda i: (0, 0)),
            pl.BlockSpec((K, D), lambda i: (0, 0)),
        ],
        out_specs=pl.BlockSpec((BT,), lambda i: (i,)),
        out_shape=jax.ShapeDtypeStruct((N,), jnp.int32),
    )(z, cbT, codebook)

    DP = 128
    codebook_pad = jnp.pad(codebook, ((0, 0), (0, DP - D)))
    zq_pad = _sc_gather(codebook_pad, idx_flat, N, D, DP)
    return idx_flat.reshape(B, T), zq_pad[:, :D].reshape(B, T, D)
